# baseline (speedup 1.0000x reference)
import functools

import jax
import jax.numpy as jnp
from jax import lax
from jax.experimental import pallas as pl
from jax.experimental.pallas import tpu as pltpu
from jax.experimental.pallas import tpu_sc as plsc

D = 32
NC = 2
NS = 16
NW = NC * NS
B = 16384
H = 200
BT = B // 128
BT_PER_W = BT // NW
HG = H // 8


def _make_lookup():
    mesh = plsc.VectorSubcoreMesh(core_axis_name="c", subcore_axis_name="s")

    @functools.partial(
        pl.kernel,
        out_type=jax.ShapeDtypeStruct((H, D // 8, BT, 8, 128), jnp.float32),
        mesh=mesh,
        scratch_types=[
            pltpu.VMEM((HG, 8, 128), jnp.int32),
            pltpu.VMEM((1024, D), jnp.float32),
            pltpu.VMEM((1024, D), jnp.float32),
            pltpu.VMEM((8, D // 8, 8, 128), jnp.float32),
            pltpu.SemaphoreType.DMA,
            pltpu.SemaphoreType.DMA,
        ],
        compiler_params=pltpu.CompilerParams(
            use_tc_tiling_on_sc=False, needs_layout_passes=False
        ),
    )
    def lookup(idx_hbm, table_hbm, out_hbm, idx_v, ga, gb, tv, gsem, osem):
        wid = lax.axis_index("s") * NC + lax.axis_index("c")
        iota = jnp.arange(16, dtype=jnp.int32)

        def fire_gathers(g, gbuf):
            for i in range(8):
                pltpu.async_copy(
                    table_hbm.at[idx_v.at[g, i]],
                    gbuf.at[pl.ds(i * 128, 128)],
                    gsem,
                )

        def drain_gathers(gbuf):
            for i in range(8):
                pltpu.make_async_copy(
                    table_hbm.at[pl.ds(0, 128)],
                    gbuf.at[pl.ds(i * 128, 128)],
                    gsem,
                ).wait()

        def drain_stores():
            for b in range(8):
                pltpu.make_async_copy(
                    tv.at[b], out_hbm.at[0, :, 0], osem
                ).wait()

        def transpose_group(gbuf):
            def tblk(blk, carry):
                base = blk * 128
                rows = [iota + (base + j16 * 16) for j16 in range(8)]
                for d in range(D):
                    cold = jnp.full((16,), d, dtype=jnp.int32)
                    for j16 in range(8):
                        v = plsc.load_gather(gbuf, [rows[j16], cold])
                        tv[blk, d // 8, d % 8, pl.ds(j16 * 16, 16)] = v
                return carry

            lax.fori_loop(0, 8, tblk, 0)

        def process(g, bt, gbuf, drain_pred):
            drain_gathers(gbuf)
            if drain_pred is None:
                drain_stores()
            else:
                @pl.when(drain_pred)
                def _():
                    drain_stores()
            transpose_group(gbuf)
            for b in range(8):
                pltpu.async_copy(
                    tv.at[b], out_hbm.at[8 * g + b, :, bt], osem
                )

        def bt_body(bt_l, carry):
            bt = wid * BT_PER_W + bt_l
            pltpu.sync_copy(idx_hbm.at[:, bt], idx_v)
            fire_gathers(0, ga)

            def pair(k, kcarry):
                fire_gathers(2 * k + 1, gb)
                process(2 * k, bt, ga, (bt_l > 0) | (k > 0))
                fire_gathers(2 * k + 2, ga)
                process(2 * k + 1, bt, gb, None)
                return kcarry

            lax.fori_loop(0, (HG - 1) // 2, pair, 0)
            process(HG - 1, bt, ga, None)
            return carry

        lax.fori_loop(0, BT_PER_W, bt_body, 0)
        drain_stores()

    return lookup


_LOOKUP = _make_lookup()


def kernel(inputs, embeddings):
    idx_p = inputs.T.reshape(HG, 8, BT, 128).transpose(0, 2, 1, 3)
    out_p = _LOOKUP(idx_p, embeddings)
    x = out_p.transpose(0, 1, 3, 2, 4).reshape(H, D, B)
    return x.transpose(2, 0, 1)
